# baseline (speedup 1.0000x reference)
import functools

import jax
import jax.numpy as jnp
from jax import lax
from jax.experimental import pallas as pl
from jax.experimental.pallas import tpu as pltpu
from jax.experimental.pallas import tpu_sc as plsc

B, C, D, H, W = 8, 128, 32, 32, 32
N = 32768
DHW = D * H * W
L = 16
P = 2048
NCHUNK = N // P
QUARTER = N // 4

_B_PER_CORE = 4
_CH_PER_W = 32


def _lerp_group_body(plane_r, base_r, fx_r, fy_r, fz_r, out_r):
    def body(s16):
        bv = base_r[pl.ds(s16, L)]
        fxv = fx_r[pl.ds(s16, L)]
        fyv = fy_r[pl.ds(s16, L)]
        fzv = fz_r[pl.ds(s16, L)]
        v000 = plsc.load_gather(plane_r, [bv])
        v001 = plsc.load_gather(plane_r, [bv + 1])
        v010 = plsc.load_gather(plane_r, [bv + W])
        v011 = plsc.load_gather(plane_r, [bv + (W + 1)])
        v100 = plsc.load_gather(plane_r, [bv + H * W])
        v101 = plsc.load_gather(plane_r, [bv + (H * W + 1)])
        v110 = plsc.load_gather(plane_r, [bv + (H * W + W)])
        v111 = plsc.load_gather(plane_r, [bv + (H * W + W + 1)])
        x00 = v000 + fxv * (v001 - v000)
        x01 = v010 + fxv * (v011 - v010)
        x10 = v100 + fxv * (v101 - v100)
        x11 = v110 + fxv * (v111 - v110)
        y0 = x00 + fyv * (x01 - x00)
        y1 = x10 + fyv * (x11 - x10)
        out_r[pl.ds(s16, L)] = y0 + fzv * (y1 - y0)
    return body


def _make_sc_kernel():
    info = plsc.get_sparse_core_info()
    nc = info.num_cores
    mesh = plsc.VectorSubcoreMesh(core_axis_name="c", subcore_axis_name="s")

    @functools.partial(
        pl.kernel,
        mesh=mesh,
        out_type=jax.ShapeDtypeStruct((B, C, N), jnp.float32),
        compiler_params=pltpu.CompilerParams(needs_layout_passes=False),
        scratch_types=[
            pltpu.VMEM((3 * P,), jnp.float32),
            pltpu.VMEM((P,), jnp.int32),
            pltpu.VMEM((P,), jnp.int32),
            pltpu.VMEM((P,), jnp.float32),
            pltpu.VMEM((P,), jnp.float32),
            pltpu.VMEM((P,), jnp.float32),
            pltpu.VMEM((P,), jnp.float32),
            pltpu.VMEM((P,), jnp.float32),
            pltpu.VMEM((P,), jnp.float32),
            pltpu.VMEM((DHW,), jnp.float32),
            pltpu.VMEM((DHW,), jnp.float32),
            pltpu.VMEM((P,), jnp.float32),
            pltpu.VMEM((P,), jnp.float32),
            pltpu.VMEM_SHARED((_B_PER_CORE, N), jnp.int32),
            pltpu.VMEM_SHARED((_B_PER_CORE, N), jnp.float32),
            pltpu.VMEM_SHARED((_B_PER_CORE, N), jnp.float32),
            pltpu.VMEM_SHARED((_B_PER_CORE, N), jnp.float32),
            pltpu.SemaphoreType.DMA,
            pltpu.SemaphoreType.DMA,
            pltpu.SemaphoreType.DMA,
        ],
    )
    def sc_devox(vox_hbm, coords_hbm, out_hbm,
                 coords_v, base0, base1, fx0, fx1, fy0, fy1, fz0, fz1,
                 plane0, plane1, outb0, outb1,
                 sp_base, sp_fx, sp_fy, sp_fz,
                 sem_pl, sem_pp, sem_out):
        core = lax.axis_index("c")
        sub = lax.axis_index("s")
        lb = sub // 4
        b = core * _B_PER_CORE + lb
        q = sub % 4
        c0 = q * _CH_PER_W

        planes = [plane0, plane1]
        basev = [base0, base1]
        fxv_ = [fx0, fx1]
        fyv_ = [fy0, fy1]
        fzv_ = [fz0, fz1]
        outs = [outb0, outb1]

        lane3 = lax.iota(jnp.int32, L) * 3
        for k in range(QUARTER // P):
            off = q * QUARTER + k * P
            pltpu.sync_copy(coords_hbm.at[b, pl.ds(off * 3, 3 * P)], coords_v)

            @plsc.parallel_loop(0, P, L, unroll=4)
            def _stage(s16):
                i3 = lane3 + s16 * 3
                x = plsc.load_gather(coords_v, [i3])
                y = plsc.load_gather(coords_v, [i3 + 1])
                z = plsc.load_gather(coords_v, [i3 + 2])
                ix = (x + 1.0) * 0.5 * (W - 1)
                iy = (y + 1.0) * 0.5 * (H - 1)
                iz = (z + 1.0) * 0.5 * (D - 1)
                bx = jnp.minimum(ix.astype(jnp.int32), W - 2)
                by = jnp.minimum(iy.astype(jnp.int32), H - 2)
                bz = jnp.minimum(iz.astype(jnp.int32), D - 2)
                base0[pl.ds(s16, L)] = (bz * H + by) * W + bx
                fx0[pl.ds(s16, L)] = ix - bx.astype(jnp.float32)
                fy0[pl.ds(s16, L)] = iy - by.astype(jnp.float32)
                fz0[pl.ds(s16, L)] = iz - bz.astype(jnp.float32)

            pltpu.sync_copy(base0, sp_base.at[lb, pl.ds(off, P)])
            pltpu.sync_copy(fx0, sp_fx.at[lb, pl.ds(off, P)])
            pltpu.sync_copy(fy0, sp_fy.at[lb, pl.ds(off, P)])
            pltpu.sync_copy(fz0, sp_fz.at[lb, pl.ds(off, P)])
        plsc.subcore_barrier()

        def issue_pp(kidx, kpar):
            off = kidx * P
            pltpu.async_copy(sp_base.at[lb, pl.ds(off, P)], basev[kpar], sem_pp)
            pltpu.async_copy(sp_fx.at[lb, pl.ds(off, P)], fxv_[kpar], sem_pp)
            pltpu.async_copy(sp_fy.at[lb, pl.ds(off, P)], fyv_[kpar], sem_pp)
            pltpu.async_copy(sp_fz.at[lb, pl.ds(off, P)], fzv_[kpar], sem_pp)

        def wait_pp(kpar):
            pltpu.make_async_copy(sp_base.at[lb, pl.ds(0, P)], basev[kpar], sem_pp).wait()
            pltpu.make_async_copy(sp_fx.at[lb, pl.ds(0, P)], fxv_[kpar], sem_pp).wait()
            pltpu.make_async_copy(sp_fy.at[lb, pl.ds(0, P)], fyv_[kpar], sem_pp).wait()
            pltpu.make_async_copy(sp_fz.at[lb, pl.ds(0, P)], fzv_[kpar], sem_pp).wait()

        def wait_plane(ppar):
            pltpu.make_async_copy(vox_hbm.at[b, c0], planes[ppar], sem_pl).wait()

        def wait_out(opar):
            pltpu.make_async_copy(outs[opar], out_hbm.at[b, c0, pl.ds(0, P)], sem_out).wait()

        pltpu.async_copy(vox_hbm.at[b, c0], planes[0], sem_pl)

        @pl.loop(0, _CH_PER_W, step=2)
        def _chan(cc):
            for ppar in range(2):
                c = cc + ppar
                wait_plane(ppar)
                if ppar == 0:
                    pltpu.async_copy(vox_hbm.at[b, c0 + cc + 1], planes[1], sem_pl)
                else:
                    @pl.when(cc < _CH_PER_W - 2)
                    def _():
                        pltpu.async_copy(vox_hbm.at[b, c0 + cc + 2], planes[0], sem_pl)

                issue_pp(0, 0)

                @pl.loop(0, NCHUNK, step=2)
                def _chunk(kk):
                    for kpar in range(2):
                        k = kk + kpar
                        wait_pp(kpar)
                        if kpar == 0:
                            issue_pp(kk + 1, 1)
                        else:
                            @pl.when(kk < NCHUNK - 2)
                            def _():
                                issue_pp(kk + 2, 0)
                        @pl.when(kk >= 2)
                        def _():
                            wait_out(kpar)
                        plsc.parallel_loop(0, P, L, unroll=4)(
                            _lerp_group_body(planes[ppar], basev[kpar],
                                             fxv_[kpar], fyv_[kpar],
                                             fzv_[kpar], outs[kpar])
                        )
                        pltpu.async_copy(
                            outs[kpar], out_hbm.at[b, c0 + c, pl.ds(k * P, P)],
                            sem_out)

                wait_out(0)
                wait_out(1)

    return sc_devox


_sc_devox = _make_sc_kernel()


@jax.jit
def kernel(vox_bcrrr, coords_bnc3):
    vox_flat = vox_bcrrr.reshape(B, C, DHW)
    coords_flat = coords_bnc3.reshape(B, 3 * N)
    return _sc_devox(vox_flat, coords_flat)


# speedup vs baseline: 1.1585x; 1.1585x over previous
import functools

import jax
import jax.numpy as jnp
from jax import lax
from jax.experimental import pallas as pl
from jax.experimental.pallas import tpu as pltpu
from jax.experimental.pallas import tpu_sc as plsc

B, C, D, H, W = 8, 128, 32, 32, 32
N = 32768
DHW = D * H * W
L = 16
P = 2048
NCHUNK = N // P
QUARTER = N // 4

_B_PER_CORE = 4
_CH_PER_W = 32


def _prep_body(c_ref, base_ref, fx_ref, fy_ref, fz_ref):
    x = c_ref[0]
    y = c_ref[1]
    z = c_ref[2]
    ix = (x + 1.0) * 0.5 * (W - 1)
    iy = (y + 1.0) * 0.5 * (H - 1)
    iz = (z + 1.0) * 0.5 * (D - 1)
    bx = jnp.clip(jnp.floor(ix), 0.0, W - 2)
    by = jnp.clip(jnp.floor(iy), 0.0, H - 2)
    bz = jnp.clip(jnp.floor(iz), 0.0, D - 2)
    fx_ref[...] = ix - bx
    fy_ref[...] = iy - by
    fz_ref[...] = iz - bz
    base_ref[...] = (
        bz.astype(jnp.int32) * (H * W)
        + by.astype(jnp.int32) * W
        + bx.astype(jnp.int32)
    )


def _prep(coords_t):
    return pl.pallas_call(
        _prep_body,
        out_shape=[
            jax.ShapeDtypeStruct((B, N), jnp.int32),
            jax.ShapeDtypeStruct((B, N), jnp.float32),
            jax.ShapeDtypeStruct((B, N), jnp.float32),
            jax.ShapeDtypeStruct((B, N), jnp.float32),
        ],
    )(coords_t)


def _lerp_group_body(plane_r, base_r, fx_r, fy_r, fz_r, out_r):
    def body(s16):
        bv = base_r[pl.ds(s16, L)]
        fxv = fx_r[pl.ds(s16, L)]
        fyv = fy_r[pl.ds(s16, L)]
        fzv = fz_r[pl.ds(s16, L)]
        v000 = plsc.load_gather(plane_r, [bv])
        v001 = plsc.load_gather(plane_r, [bv + 1])
        v010 = plsc.load_gather(plane_r, [bv + W])
        v011 = plsc.load_gather(plane_r, [bv + (W + 1)])
        v100 = plsc.load_gather(plane_r, [bv + H * W])
        v101 = plsc.load_gather(plane_r, [bv + (H * W + 1)])
        v110 = plsc.load_gather(plane_r, [bv + (H * W + W)])
        v111 = plsc.load_gather(plane_r, [bv + (H * W + W + 1)])
        x00 = v000 + fxv * (v001 - v000)
        x01 = v010 + fxv * (v011 - v010)
        x10 = v100 + fxv * (v101 - v100)
        x11 = v110 + fxv * (v111 - v110)
        y0 = x00 + fyv * (x01 - x00)
        y1 = x10 + fyv * (x11 - x10)
        out_r[pl.ds(s16, L)] = y0 + fzv * (y1 - y0)
    return body


def _make_sc_kernel():
    info = plsc.get_sparse_core_info()
    nc = info.num_cores
    mesh = plsc.VectorSubcoreMesh(core_axis_name="c", subcore_axis_name="s")

    @functools.partial(
        pl.kernel,
        mesh=mesh,
        out_type=jax.ShapeDtypeStruct((B, C, N), jnp.float32),
        compiler_params=pltpu.CompilerParams(needs_layout_passes=False),
        scratch_types=[
            pltpu.VMEM((P,), jnp.int32),
            pltpu.VMEM((P,), jnp.int32),
            pltpu.VMEM((P,), jnp.float32),
            pltpu.VMEM((P,), jnp.float32),
            pltpu.VMEM((P,), jnp.float32),
            pltpu.VMEM((P,), jnp.float32),
            pltpu.VMEM((P,), jnp.float32),
            pltpu.VMEM((P,), jnp.float32),
            pltpu.VMEM((DHW,), jnp.float32),
            pltpu.VMEM((DHW,), jnp.float32),
            pltpu.VMEM((P,), jnp.float32),
            pltpu.VMEM((P,), jnp.float32),
            pltpu.VMEM_SHARED((_B_PER_CORE, N), jnp.int32),
            pltpu.VMEM_SHARED((_B_PER_CORE, N), jnp.float32),
            pltpu.VMEM_SHARED((_B_PER_CORE, N), jnp.float32),
            pltpu.VMEM_SHARED((_B_PER_CORE, N), jnp.float32),
            pltpu.SemaphoreType.DMA,
            pltpu.SemaphoreType.DMA,
            pltpu.SemaphoreType.DMA,
        ],
    )
    def sc_devox(vox_hbm, base_hbm, fx_hbm, fy_hbm, fz_hbm, out_hbm,
                 base0, base1, fx0, fx1, fy0, fy1, fz0, fz1,
                 plane0, plane1, outb0, outb1,
                 sp_base, sp_fx, sp_fy, sp_fz,
                 sem_pl, sem_pp, sem_out):
        core = lax.axis_index("c")
        sub = lax.axis_index("s")
        lb = sub // 4
        b = core * _B_PER_CORE + lb
        q = sub % 4
        c0 = q * _CH_PER_W

        planes = [plane0, plane1]
        basev = [base0, base1]
        fxv_ = [fx0, fx1]
        fyv_ = [fy0, fy1]
        fzv_ = [fz0, fz1]
        outs = [outb0, outb1]

        off = q * QUARTER
        pltpu.sync_copy(base_hbm.at[b, pl.ds(off, QUARTER)],
                        sp_base.at[lb, pl.ds(off, QUARTER)])
        pltpu.sync_copy(fx_hbm.at[b, pl.ds(off, QUARTER)],
                        sp_fx.at[lb, pl.ds(off, QUARTER)])
        pltpu.sync_copy(fy_hbm.at[b, pl.ds(off, QUARTER)],
                        sp_fy.at[lb, pl.ds(off, QUARTER)])
        pltpu.sync_copy(fz_hbm.at[b, pl.ds(off, QUARTER)],
                        sp_fz.at[lb, pl.ds(off, QUARTER)])
        plsc.subcore_barrier()

        def issue_pp(kidx, kpar):
            off = kidx * P
            pltpu.async_copy(sp_base.at[lb, pl.ds(off, P)], basev[kpar], sem_pp)
            pltpu.async_copy(sp_fx.at[lb, pl.ds(off, P)], fxv_[kpar], sem_pp)
            pltpu.async_copy(sp_fy.at[lb, pl.ds(off, P)], fyv_[kpar], sem_pp)
            pltpu.async_copy(sp_fz.at[lb, pl.ds(off, P)], fzv_[kpar], sem_pp)

        def wait_pp(kpar):
            pltpu.make_async_copy(sp_base.at[lb, pl.ds(0, P)], basev[kpar], sem_pp).wait()
            pltpu.make_async_copy(sp_fx.at[lb, pl.ds(0, P)], fxv_[kpar], sem_pp).wait()
            pltpu.make_async_copy(sp_fy.at[lb, pl.ds(0, P)], fyv_[kpar], sem_pp).wait()
            pltpu.make_async_copy(sp_fz.at[lb, pl.ds(0, P)], fzv_[kpar], sem_pp).wait()

        def wait_plane(ppar):
            pltpu.make_async_copy(vox_hbm.at[b, c0], planes[ppar], sem_pl).wait()

        def wait_out(opar):
            pltpu.make_async_copy(outs[opar], out_hbm.at[b, c0, pl.ds(0, P)], sem_out).wait()

        pltpu.async_copy(vox_hbm.at[b, c0], planes[0], sem_pl)

        @pl.loop(0, _CH_PER_W, step=2)
        def _chan(cc):
            for ppar in range(2):
                c = cc + ppar
                wait_plane(ppar)
                if ppar == 0:
                    pltpu.async_copy(vox_hbm.at[b, c0 + cc + 1], planes[1], sem_pl)
                else:
                    @pl.when(cc < _CH_PER_W - 2)
                    def _():
                        pltpu.async_copy(vox_hbm.at[b, c0 + cc + 2], planes[0], sem_pl)

                issue_pp(0, 0)

                @pl.loop(0, NCHUNK, step=2)
                def _chunk(kk):
                    for kpar in range(2):
                        k = kk + kpar
                        wait_pp(kpar)
                        if kpar == 0:
                            issue_pp(kk + 1, 1)
                        else:
                            @pl.when(kk < NCHUNK - 2)
                            def _():
                                issue_pp(kk + 2, 0)
                        @pl.when(kk >= 2)
                        def _():
                            wait_out(kpar)
                        plsc.parallel_loop(0, P, L, unroll=4)(
                            _lerp_group_body(planes[ppar], basev[kpar],
                                             fxv_[kpar], fyv_[kpar],
                                             fzv_[kpar], outs[kpar])
                        )
                        pltpu.async_copy(
                            outs[kpar], out_hbm.at[b, c0 + c, pl.ds(k * P, P)],
                            sem_out)

                wait_out(0)
                wait_out(1)

    return sc_devox


_sc_devox = _make_sc_kernel()


@jax.jit
def kernel(vox_bcrrr, coords_bnc3):
    vox_flat = vox_bcrrr.reshape(B, C, DHW)
    coords_t = jnp.transpose(coords_bnc3, (2, 0, 1))
    base, fx, fy, fz = _prep(coords_t)
    return _sc_devox(vox_flat, base, fx, fy, fz)
